# baseline (speedup 1.0000x reference)
import functools

import jax
import jax.numpy as jnp
from jax import lax
from jax.experimental import pallas as pl
from jax.experimental.pallas import tpu as pltpu
from jax.experimental.pallas import tpu_sc as plsc

BATCH = 16384
EMBED = 32
NW = 32
B_PER_W = BATCH // NW
N_SEG = 4
SEG = B_PER_W // N_SEG
N_CHUNK = B_PER_W // 16

_mesh = plsc.VectorSubcoreMesh(core_axis_name="c", subcore_axis_name="s")


@functools.partial(
    pl.kernel,
    mesh=_mesh,
    out_type=jax.ShapeDtypeStruct((BATCH,), jnp.float32),
    scratch_types=[
        pltpu.VMEM((N_SEG, SEG), jnp.int32),
        pltpu.VMEM((N_SEG, SEG), jnp.int32),
        pltpu.VMEM((B_PER_W, EMBED), jnp.float32),
        pltpu.VMEM((B_PER_W, EMBED), jnp.float32),
        pltpu.VMEM((4, 16), jnp.float32),
        pltpu.VMEM((16,), jnp.float32),
        pltpu.VMEM((B_PER_W,), jnp.float32),
        pltpu.SemaphoreType.DMA,
    ],
    compiler_params=pltpu.CompilerParams(needs_layout_passes=False,
                                         use_tc_tiling_on_sc=False),
)
def _sc_fwd(users_hbm, movies_hbm, ut_hbm, mt_hbm, wb_hbm, bb_hbm, out_hbm,
            uidx, midx, urows, mrows, wv, bv, outv, sem):
    wid = lax.axis_index("s") * 2 + lax.axis_index("c")
    seg_base = wid * N_SEG

    pltpu.sync_copy(users_hbm.at[pl.ds(seg_base, N_SEG)], uidx)
    pltpu.sync_copy(movies_hbm.at[pl.ds(seg_base, N_SEG)], midx)
    pltpu.sync_copy(wb_hbm, wv)
    pltpu.sync_copy(bb_hbm, bv)

    copies = []
    for j in range(N_SEG):
        copies.append(pltpu.async_copy(
            ut_hbm.at[uidx.at[j]], urows.at[pl.ds(j * SEG, SEG)], sem))
        copies.append(pltpu.async_copy(
            mt_hbm.at[midx.at[j]], mrows.at[pl.ds(j * SEG, SEG)], sem))
    for cp in copies:
        cp.wait()

    w0 = wv[0, pl.ds(0, 16)]
    w1 = wv[1, pl.ds(0, 16)]
    w2 = wv[2, pl.ds(0, 16)]
    w3 = wv[3, pl.ds(0, 16)]
    bvec = bv[pl.ds(0, 16)]
    lane15 = lax.iota(jnp.int32, 16) == 15

    def row_body(i, carry):
        u0 = urows[i, pl.ds(0, 16)]
        u1 = urows[i, pl.ds(16, 16)]
        m0 = mrows[i, pl.ds(0, 16)]
        m1 = mrows[i, pl.ds(16, 16)]
        s = (u0 * w0 + u1 * w1) + (m0 * w2 + m1 * w3) + bvec
        csum = jnp.cumsum(s)
        plsc.store_scatter(outv, [jnp.full((16,), i, jnp.int32)], csum,
                           mask=lane15)
        return carry

    lax.fori_loop(0, B_PER_W, row_body, 0, unroll=8)

    pltpu.sync_copy(outv, out_hbm.at[pl.ds(wid * B_PER_W, B_PER_W)])


def kernel(users, movies, user_table, movie_table, W, b):
    users2d = users.astype(jnp.int32).reshape(NW * N_SEG, SEG)
    movies2d = movies.astype(jnp.int32).reshape(NW * N_SEG, SEG)
    wb = W.reshape(4, 16)
    bb = jnp.broadcast_to(b.reshape(1) / 16.0, (16,))
    out = _sc_fwd(users2d, movies2d, user_table, movie_table, wb, bb)
    return out.reshape(BATCH, 1)


# speedup vs baseline: 1.4857x; 1.4857x over previous
import functools

import jax
import jax.numpy as jnp
from jax import lax
from jax.experimental import pallas as pl
from jax.experimental.pallas import tpu as pltpu
from jax.experimental.pallas import tpu_sc as plsc

BATCH = 16384
EMBED = 32
NW = 32
B_PER_W = BATCH // NW
CH = 128
N_CH = B_PER_W // CH

_mesh = plsc.VectorSubcoreMesh(core_axis_name="c", subcore_axis_name="s")


@functools.partial(
    pl.kernel,
    mesh=_mesh,
    out_type=jax.ShapeDtypeStruct((BATCH,), jnp.float32),
    scratch_types=[
        pltpu.VMEM((B_PER_W,), jnp.int32),
        pltpu.VMEM((B_PER_W,), jnp.int32),
        pltpu.VMEM((2, CH, EMBED), jnp.float32),
        pltpu.VMEM((2, CH, EMBED), jnp.float32),
        pltpu.VMEM((4, 16), jnp.float32),
        pltpu.VMEM((16,), jnp.float32),
        pltpu.VMEM((B_PER_W,), jnp.float32),
        pltpu.SemaphoreType.DMA,
        pltpu.SemaphoreType.DMA,
    ],
    compiler_params=pltpu.CompilerParams(needs_layout_passes=False),
)
def _sc_fwd(users_hbm, movies_hbm, ut_hbm, mt_hbm, wb_hbm, bb_hbm, out_hbm,
            uidx, midx, urows, mrows, wv, bv, outv,
            sem_u, sem_m):
    wid = lax.axis_index("s") * 2 + lax.axis_index("c")
    base = wid * B_PER_W

    pltpu.sync_copy(users_hbm.at[pl.ds(base, B_PER_W)], uidx)
    pltpu.sync_copy(movies_hbm.at[pl.ds(base, B_PER_W)], midx)
    pltpu.sync_copy(wb_hbm, wv)
    pltpu.sync_copy(bb_hbm, bv)

    def fire_chunk(c, buf):
        def fetch_body(g, carry):
            uvec = uidx[pl.ds(c * CH + g * 16, 16)]
            mvec = midx[pl.ds(c * CH + g * 16, 16)]
            for j in range(16):
                pltpu.async_copy(ut_hbm.at[pl.ds(uvec[j], 1)],
                                 urows.at[buf, pl.ds(g * 16 + j, 1)], sem_u)
                pltpu.async_copy(mt_hbm.at[pl.ds(mvec[j], 1)],
                                 mrows.at[buf, pl.ds(g * 16 + j, 1)], sem_m)
            return carry
        lax.fori_loop(0, CH // 16, fetch_body, 0)

    def drain_chunk(buf):
        pltpu.make_async_copy(ut_hbm.at[pl.ds(0, CH)], urows.at[buf],
                              sem_u).wait()
        pltpu.make_async_copy(mt_hbm.at[pl.ds(0, CH)], mrows.at[buf],
                              sem_m).wait()

    w0 = wv[0, pl.ds(0, 16)]
    w1 = wv[1, pl.ds(0, 16)]
    w2 = wv[2, pl.ds(0, 16)]
    w3 = wv[3, pl.ds(0, 16)]
    bvec = bv[pl.ds(0, 16)]
    lane15 = lax.iota(jnp.int32, 16) == 15

    def compute_chunk(c, buf):
        def row_body(i, carry):
            u0 = urows[buf, i, pl.ds(0, 16)]
            u1 = urows[buf, i, pl.ds(16, 16)]
            m0 = mrows[buf, i, pl.ds(0, 16)]
            m1 = mrows[buf, i, pl.ds(16, 16)]
            s = (u0 * w0 + u1 * w1) + (m0 * w2 + m1 * w3) + bvec
            csum = jnp.cumsum(s)
            plsc.store_scatter(outv,
                               [jnp.full((16,), c * CH + i, jnp.int32)],
                               csum, mask=lane15)
            return carry
        lax.fori_loop(0, CH, row_body, 0, unroll=8)

    fire_chunk(0, 0)
    for c in range(N_CH):
        if c + 1 < N_CH:
            fire_chunk(c + 1, (c + 1) % 2)
        drain_chunk(c % 2)
        compute_chunk(c, c % 2)

    pltpu.sync_copy(outv, out_hbm.at[pl.ds(base, B_PER_W)])


def kernel(users, movies, user_table, movie_table, W, b):
    users1d = users.astype(jnp.int32)
    movies1d = movies.astype(jnp.int32)
    wb = W.reshape(4, 16)
    bb = jnp.broadcast_to(b.reshape(1) / 16.0, (16,))
    out = _sc_fwd(users1d, movies1d, user_table, movie_table, wb, bb)
    return out.reshape(BATCH, 1)


# speedup vs baseline: 3.6855x; 2.4806x over previous
import functools

import jax
import jax.numpy as jnp
from jax import lax
from jax.experimental import pallas as pl
from jax.experimental.pallas import tpu as pltpu
from jax.experimental.pallas import tpu_sc as plsc

BATCH = 16384
EMBED = 32
NROWS = 1000000
BLK = 32768
SROWS = 7936
SFLAT = SROWS * 128
R0 = 786432
R1 = 983040
HI_CHUNKS = (R1 - R0) // 128
HI_FLAT = R1 - R0
NW = 32
B_PER_W = BATCH // NW
N_SEG = 4
SEG = B_PER_W // N_SEG
CH_PER_TILE = HI_CHUNKS // NW

_mesh = plsc.VectorSubcoreMesh(core_axis_name="c", subcore_axis_name="s")


def _tc_scores(ut_ref, mt_ref, w_ref, su_ref, sm_ref):
    wu = w_ref[0, :]
    wm = w_ref[1, :]
    su = jnp.sum(ut_ref[...] * wu[:, None], axis=0)
    sm = jnp.sum(mt_ref[...] * wm[:, None], axis=0)
    su_ref[...] = su.reshape(BLK // 128, 128)
    sm_ref[...] = sm.reshape(BLK // 128, 128)


def _tc_block(g):
    return jnp.where(g < 24, g, 30)


_scores = pl.pallas_call(
    _tc_scores,
    grid=(25,),
    in_specs=[
        pl.BlockSpec((EMBED, BLK), lambda g: (0, _tc_block(g))),
        pl.BlockSpec((EMBED, BLK), lambda g: (0, _tc_block(g))),
        pl.BlockSpec((2, EMBED), lambda g: (0, 0)),
    ],
    out_specs=[
        pl.BlockSpec((BLK // 128, 128), lambda g: (_tc_block(g), 0)),
        pl.BlockSpec((BLK // 128, 128), lambda g: (_tc_block(g), 0)),
    ],
    out_shape=[
        jax.ShapeDtypeStruct((SROWS, 128), jnp.float32),
        jax.ShapeDtypeStruct((SROWS, 128), jnp.float32),
    ],
)


@functools.partial(
    pl.kernel,
    mesh=_mesh,
    out_type=[
        jax.ShapeDtypeStruct((HI_FLAT,), jnp.float32),
        jax.ShapeDtypeStruct((HI_FLAT,), jnp.float32),
    ],
    scratch_types=[
        pltpu.VMEM((EMBED, 128), jnp.float32),
        pltpu.VMEM((EMBED, 128), jnp.float32),
        pltpu.VMEM((2 * EMBED, 16), jnp.float32),
        pltpu.VMEM((128,), jnp.float32),
        pltpu.VMEM((128,), jnp.float32),
    ],
    compiler_params=pltpu.CompilerParams(needs_layout_passes=False),
)
def _sc_sweep(ut_hbm, mt_hbm, wb_hbm, su_hbm, sm_hbm,
              ublk, mblk, wv, sou, som):
    wid = lax.axis_index("s") * 2 + lax.axis_index("c")
    pltpu.sync_copy(wb_hbm, wv)

    def chunk_body(k, carry):
        q = wid * CH_PER_TILE + k
        col = R0 + q * 128
        pltpu.sync_copy(ut_hbm.at[pl.ds(0, EMBED), pl.ds(col, 128)], ublk)
        pltpu.sync_copy(mt_hbm.at[pl.ds(0, EMBED), pl.ds(col, 128)], mblk)

        def grp_body(g, carry2):
            off = g * 16
            pu = [jnp.zeros((16,), jnp.float32) for _ in range(4)]
            pm = [jnp.zeros((16,), jnp.float32) for _ in range(4)]
            for d in range(EMBED):
                pu[d % 4] = pu[d % 4] + ublk[d, pl.ds(off, 16)] * wv[d]
                pm[d % 4] = pm[d % 4] + mblk[d, pl.ds(off, 16)] * wv[EMBED + d]
            sou[pl.ds(off, 16)] = (pu[0] + pu[1]) + (pu[2] + pu[3])
            som[pl.ds(off, 16)] = (pm[0] + pm[1]) + (pm[2] + pm[3])
            return carry2

        lax.fori_loop(0, 8, grp_body, 0)
        pltpu.sync_copy(sou, su_hbm.at[pl.ds(q * 128, 128)])
        pltpu.sync_copy(som, sm_hbm.at[pl.ds(q * 128, 128)])
        return carry

    lax.fori_loop(0, CH_PER_TILE, chunk_body, 0)


@functools.partial(
    pl.kernel,
    mesh=_mesh,
    out_type=jax.ShapeDtypeStruct((BATCH,), jnp.float32),
    scratch_types=[
        pltpu.VMEM((N_SEG, SEG), jnp.int32),
        pltpu.VMEM((N_SEG, SEG), jnp.int32),
        pltpu.VMEM((N_SEG, SEG), jnp.int32),
        pltpu.VMEM((N_SEG, SEG), jnp.int32),
        pltpu.VMEM((B_PER_W,), jnp.float32),
        pltpu.VMEM((B_PER_W,), jnp.float32),
        pltpu.VMEM((B_PER_W,), jnp.float32),
        pltpu.VMEM((B_PER_W,), jnp.float32),
        pltpu.VMEM((16,), jnp.float32),
        pltpu.VMEM((B_PER_W,), jnp.float32),
        pltpu.SemaphoreType.DMA,
    ],
    compiler_params=pltpu.CompilerParams(needs_layout_passes=False),
)
def _sc_gather(users_hbm, movies_hbm, su_hbm, sm_hbm, suh_hbm, smh_hbm,
               bb_hbm, out_hbm,
               uidx, midx, uidxh, midxh, ug, mg, ugh, mgh, bv, outv, sem):
    wid = lax.axis_index("s") * 2 + lax.axis_index("c")
    seg_base = wid * N_SEG

    pltpu.sync_copy(users_hbm.at[pl.ds(seg_base, N_SEG)], uidx)
    pltpu.sync_copy(movies_hbm.at[pl.ds(seg_base, N_SEG)], midx)
    pltpu.sync_copy(bb_hbm, bv)

    copies = []
    for j in range(N_SEG):
        copies.append(pltpu.async_copy(
            su_hbm.at[uidx.at[j]], ug.at[pl.ds(j * SEG, SEG)], sem))
        copies.append(pltpu.async_copy(
            sm_hbm.at[midx.at[j]], mg.at[pl.ds(j * SEG, SEG)], sem))

    zero = jnp.zeros((16,), jnp.int32)
    himax = jnp.full((16,), HI_FLAT - 1, jnp.int32)
    for j in range(N_SEG):
        for g in range(SEG // 16):
            uo = uidx[j, pl.ds(g * 16, 16)]
            mo = midx[j, pl.ds(g * 16, 16)]
            uidxh[j, pl.ds(g * 16, 16)] = jnp.minimum(
                jnp.maximum(uo - R0, zero), himax)
            midxh[j, pl.ds(g * 16, 16)] = jnp.minimum(
                jnp.maximum(mo - R0, zero), himax)
    for j in range(N_SEG):
        copies.append(pltpu.async_copy(
            suh_hbm.at[uidxh.at[j]], ugh.at[pl.ds(j * SEG, SEG)], sem))
        copies.append(pltpu.async_copy(
            smh_hbm.at[midxh.at[j]], mgh.at[pl.ds(j * SEG, SEG)], sem))
    for cp in copies:
        cp.wait()

    bvec = bv[pl.ds(0, 16)]
    r0v = jnp.full((16,), R0, jnp.int32)
    r1v = jnp.full((16,), R1, jnp.int32)

    def chunk_body(c, carry):
        off = c * 16
        uo = uidx[c // (SEG // 16), pl.ds((c % (SEG // 16)) * 16, 16)]
        mo = midx[c // (SEG // 16), pl.ds((c % (SEG // 16)) * 16, 16)]
        u_hi = (uo >= r0v) & (uo < r1v)
        m_hi = (mo >= r0v) & (mo < r1v)
        uval = jnp.where(u_hi, ugh[pl.ds(off, 16)], ug[pl.ds(off, 16)])
        mval = jnp.where(m_hi, mgh[pl.ds(off, 16)], mg[pl.ds(off, 16)])
        outv[pl.ds(off, 16)] = uval + mval + bvec
        return carry

    lax.fori_loop(0, B_PER_W // 16, chunk_body, 0)

    pltpu.sync_copy(outv, out_hbm.at[pl.ds(wid * B_PER_W, B_PER_W)])


def kernel(users, movies, user_table, movie_table, W, b):
    users2d = users.astype(jnp.int32).reshape(NW * N_SEG, SEG)
    movies2d = movies.astype(jnp.int32).reshape(NW * N_SEG, SEG)
    ut_t = user_table.T
    mt_t = movie_table.T
    w2 = W.reshape(2, EMBED)
    wb = jnp.broadcast_to(W.reshape(2 * EMBED, 1), (2 * EMBED, 16))
    su2d, sm2d = _scores(ut_t, mt_t, w2)
    suh, smh = _sc_sweep(ut_t, mt_t, wb)
    su = su2d.reshape(SFLAT)
    sm = sm2d.reshape(SFLAT)
    bb = jnp.broadcast_to(b.reshape(1), (16,))
    out = _sc_gather(users2d, movies2d, su, sm, suh, smh, bb)
    return out.reshape(BATCH, 1)


# speedup vs baseline: 8.4399x; 2.2900x over previous
import functools

import jax
import jax.numpy as jnp
from jax import lax
from jax.experimental import pallas as pl
from jax.experimental.pallas import tpu as pltpu
from jax.experimental.pallas import tpu_sc as plsc

BATCH = 16384
EMBED = 32
NROWS = 1000000
BLK = 32768
GRID = (NROWS + BLK - 1) // BLK
SROWS = 7936
SFLAT = SROWS * 128
NW = 32
B_PER_W = BATCH // NW
N_SEG = 4
SEG = B_PER_W // N_SEG

_mesh = plsc.VectorSubcoreMesh(core_axis_name="c", subcore_axis_name="s")


def _tc_scores(ut_ref, mt_ref, w_ref, su_ref, sm_ref):
    wu = w_ref[0, :]
    wm = w_ref[1, :]
    su = jnp.sum(ut_ref[...] * wu[:, None], axis=0)
    sm = jnp.sum(mt_ref[...] * wm[:, None], axis=0)
    su_ref[...] = su.reshape(BLK // 128, 128)
    sm_ref[...] = sm.reshape(BLK // 128, 128)


_scores = pl.pallas_call(
    _tc_scores,
    grid=(GRID,),
    in_specs=[
        pl.BlockSpec((EMBED, BLK), lambda j: (0, j)),
        pl.BlockSpec((EMBED, BLK), lambda j: (0, j)),
        pl.BlockSpec((2, EMBED), lambda j: (0, 0)),
    ],
    out_specs=[
        pl.BlockSpec((BLK // 128, 128), lambda j: (j, 0)),
        pl.BlockSpec((BLK // 128, 128), lambda j: (j, 0)),
    ],
    out_shape=[
        jax.ShapeDtypeStruct((SROWS, 128), jnp.float32),
        jax.ShapeDtypeStruct((SROWS, 128), jnp.float32),
    ],
)


@functools.partial(
    pl.kernel,
    mesh=_mesh,
    out_type=jax.ShapeDtypeStruct((BATCH,), jnp.float32),
    scratch_types=[
        pltpu.VMEM((N_SEG, SEG), jnp.int32),
        pltpu.VMEM((N_SEG, SEG), jnp.int32),
        pltpu.VMEM((B_PER_W,), jnp.float32),
        pltpu.VMEM((B_PER_W,), jnp.float32),
        pltpu.VMEM((16,), jnp.float32),
        pltpu.VMEM((B_PER_W,), jnp.float32),
        pltpu.SemaphoreType.DMA,
    ],
    compiler_params=pltpu.CompilerParams(needs_layout_passes=False),
)
def _sc_gather(users_hbm, movies_hbm, su_hbm, sm_hbm, bb_hbm, out_hbm,
               uidx, midx, ug, mg, bv, outv, sem):
    cid = lax.axis_index("c")
    sid = lax.axis_index("s")
    wid = sid * 2 + cid
    seg_base = wid * N_SEG

    pltpu.sync_copy(users_hbm.at[pl.ds(seg_base, N_SEG)], uidx)
    pltpu.sync_copy(movies_hbm.at[pl.ds(seg_base, N_SEG)], midx)
    pltpu.sync_copy(bb_hbm, bv)

    copies = []
    for j in range(N_SEG):
        copies.append(pltpu.async_copy(
            su_hbm.at[uidx.at[j]], ug.at[pl.ds(j * SEG, SEG)], sem))
        copies.append(pltpu.async_copy(
            sm_hbm.at[midx.at[j]], mg.at[pl.ds(j * SEG, SEG)], sem))
    for cp in copies:
        cp.wait()

    bvec = bv[pl.ds(0, 16)]

    def chunk_body(c, carry):
        off = c * 16
        outv[pl.ds(off, 16)] = ug[pl.ds(off, 16)] + mg[pl.ds(off, 16)] + bvec
        return carry

    lax.fori_loop(0, B_PER_W // 16, chunk_body, 0)

    pltpu.sync_copy(outv, out_hbm.at[pl.ds(wid * B_PER_W, B_PER_W)])


def kernel(users, movies, user_table, movie_table, W, b):
    users2d = users.astype(jnp.int32).reshape(NW * N_SEG, SEG)
    movies2d = movies.astype(jnp.int32).reshape(NW * N_SEG, SEG)
    ut_t = user_table.T
    mt_t = movie_table.T
    w2 = W.reshape(2, EMBED)
    su2d, sm2d = _scores(ut_t, mt_t, w2)
    su = su2d.reshape(SFLAT)
    sm = sm2d.reshape(SFLAT)
    bb = jnp.broadcast_to(b.reshape(1), (16,))
    out = _sc_gather(users2d, movies2d, su, sm, bb)
    return out.reshape(BATCH, 1)
